# Initial kernel scaffold; baseline (speedup 1.0000x reference)
import jax
import jax.numpy as jnp
from jax.experimental import pallas as pl


def kernel(x, edge_index, mode, W2a, b2a, W2b, b2b, W2c, b2c, Wd1, bd1, Wd2, bd2, Wd3, bd3, Wd4, bd4):
    raise NotImplementedError("write your pallas kernel here")


# speedup vs baseline: 18.8958x; 18.8958x over previous
import functools

import jax
import jax.numpy as jnp
from jax import lax
from jax.experimental import pallas as pl
from jax.experimental.pallas import tpu as pltpu
from jax.experimental.pallas import tpu_sc as plsc

N = 10000
E = 320000
D = 128
H = 128
T = 3

NC = 2
NS = 16
NW = NC * NS
EPW = E // NW
CH = 80
NCHUNK = EPW // CH
NV = CH // 16
NPAD = 10240
RPT = NPAD // NS

def _rsqrt16(q):
    i = plsc.bitcast(q, jnp.int32)
    i = jnp.int32(0x5F3759DF) - lax.shift_right_logical(i, 1)
    y = plsc.bitcast(i, jnp.float32)
    for _ in range(3):
        y = y * (1.5 - 0.5 * q * y * y)
    return y


def _build_edge_sc():
    mesh = plsc.VectorSubcoreMesh(core_axis_name="c", subcore_axis_name="s")
    return functools.partial(
        pl.kernel,
        mesh=mesh,
        compiler_params=pltpu.CompilerParams(needs_layout_passes=False),
        out_type=jax.ShapeDtypeStruct((6, NC, 1, NPAD), jnp.float32),
        scratch_types=[
            pltpu.VMEM((NCHUNK, CH), jnp.int32),
            pltpu.VMEM((NCHUNK, CH), jnp.int32),
            pltpu.VMEM((N * 4,), jnp.float32),
            pltpu.VMEM((CH,), jnp.float32),
            pltpu.VMEM((CH,), jnp.float32),
            pltpu.VMEM((CH,), jnp.float32),
            pltpu.VMEM((CH,), jnp.float32),
            pltpu.VMEM((CH,), jnp.float32),
            pltpu.VMEM((CH,), jnp.float32),
            pltpu.VMEM_SHARED((NPAD,), jnp.float32),
            pltpu.VMEM_SHARED((NPAD,), jnp.float32),
            pltpu.VMEM_SHARED((NPAD,), jnp.float32),
            pltpu.VMEM_SHARED((NPAD,), jnp.float32),
            pltpu.VMEM_SHARED((NPAD,), jnp.float32),
            pltpu.VMEM_SHARED((NPAD,), jnp.float32),
        ],
    )(_edge_sc_body)


_EDGE_SC = None


def _edge_sc(row3, col3, feat_flat, zeros):
    global _EDGE_SC
    if _EDGE_SC is None:
        _EDGE_SC = _build_edge_sc()
    return _EDGE_SC(row3, col3, feat_flat, zeros)


def _edge_sc_body(row_hbm, col_hbm, feat_hbm, zero_hbm, out_hbm,
                  row_v, col_v, feat_v, m0, m1, m2, m3, m4, m5,
                  a0, a1, a2, a3, a4, a5):
    c = lax.axis_index("c")
    s = lax.axis_index("s")
    wid = c * NS + s
    accs = (a0, a1, a2, a3, a4, a5)

    pltpu.sync_copy(row_hbm.at[wid], row_v)
    pltpu.sync_copy(col_hbm.at[wid], col_v)
    pltpu.sync_copy(feat_hbm, feat_v)
    for acc in accs:
        pltpu.sync_copy(zero_hbm.at[0, pl.ds(s * RPT, RPT)],
                        acc.at[pl.ds(s * RPT, RPT)])

    ones16 = jnp.full((16,), 1.0, jnp.float32)
    for g in range(NV):
        m5[pl.ds(g * 16, 16)] = ones16

    plsc.subcore_barrier()

    def chunk_body(j, carry):
        for v in range(NV):
            r16 = row_v[j, pl.ds(v * 16, 16)]
            c16 = col_v[j, pl.ds(v * 16, 16)]
            bs = lax.shift_left(r16, 2)
            bd = lax.shift_left(c16, 2)
            s0 = plsc.load_gather(feat_v, [bs])
            s1 = plsc.load_gather(feat_v, [bs + 1])
            s2 = plsc.load_gather(feat_v, [bs + 2])
            sf = plsc.load_gather(feat_v, [bs + 3])
            d0 = plsc.load_gather(feat_v, [bd])
            d1 = plsc.load_gather(feat_v, [bd + 1])
            d2 = plsc.load_gather(feat_v, [bd + 2])
            u0 = d0 - s0
            u1 = d1 - s1
            u2 = d2 - s2
            q = u0 * u0 + u1 * u1 + u2 * u2
            nrm = jnp.where(q > 0.0, q * _rsqrt16(q), 0.0)
            sl = pl.ds(v * 16, 16)
            m0[sl] = s0
            m1[sl] = s1
            m2[sl] = s2
            m3[sl] = sf
            m4[sl] = nrm
        idx = col_v.at[j]
        pltpu.sync_copy(m0, a0.at[idx], add=True)
        pltpu.sync_copy(m1, a1.at[idx], add=True)
        pltpu.sync_copy(m2, a2.at[idx], add=True)
        pltpu.sync_copy(m3, a3.at[idx], add=True)
        pltpu.sync_copy(m4, a4.at[idx], add=True)
        pltpu.sync_copy(m5, a5.at[idx], add=True)
        return carry

    lax.fori_loop(0, NCHUNK, chunk_body, 0)

    plsc.subcore_barrier()

    for cc, acc in enumerate(accs):
        pltpu.sync_copy(acc.at[pl.ds(s * RPT, RPT)],
                        out_hbm.at[cc, c, 0, pl.ds(s * RPT, RPT)])


def _mean_aggr(xb, p_ref, rows):
    A = p_ref[0] + p_ref[1]
    cnt = A[:, 5:6]
    denom = jnp.maximum(cnt, 1.0)
    disp = cnt * xb[:, 0:3] - A[:, 0:3]
    fagg = cnt * xb[:, 127:128] - A[:, 3:4]
    nrm = A[:, 4:5]
    aggr = jnp.concatenate(
        [disp, nrm, fagg, jnp.zeros((rows, 3), jnp.float32)], axis=1)
    return aggr / denom


def _node_mlp(xb, aggr, wtop_ref, wpad_ref, b1_ref, w2_ref, b2_ref, w3_ref,
              b3_ref):
    h = jnp.dot(aggr, wtop_ref[:], preferred_element_type=jnp.float32)
    h = h + jnp.dot(xb, wpad_ref[:], preferred_element_type=jnp.float32)
    h = jnp.maximum(h + b1_ref[:], 0.0)
    h = jnp.maximum(
        jnp.dot(h, w2_ref[:], preferred_element_type=jnp.float32) + b2_ref[:],
        0.0)
    res = jnp.dot(h, w3_ref[:], preferred_element_type=jnp.float32) + b3_ref[:]
    return xb + jnp.maximum(res, 0.0)


_R = 1000


def _tc_layer(x, P, wtop, wpad, b1, w2, b2, w3, b3):
    def body(x_ref, p_ref, wtop_ref, wpad_ref, b1_ref, w2_ref, b2_ref, w3_ref,
             b3_ref, xo_ref, feat_ref):
        xb = x_ref[:]
        aggr = _mean_aggr(xb, p_ref, _R)
        xn = _node_mlp(xb, aggr, wtop_ref, wpad_ref, b1_ref, w2_ref, b2_ref,
                       w3_ref, b3_ref)
        xo_ref[:] = xn
        feat_ref[:] = jnp.concatenate([xn[:, 0:3], xn[:, 127:128]], axis=1)

    full = lambda shape: pl.BlockSpec(shape, lambda i: tuple(0 for _ in shape))
    return pl.pallas_call(
        body,
        grid=(N // _R,),
        in_specs=[
            pl.BlockSpec((_R, D), lambda i: (i, 0)),
            pl.BlockSpec((NC, _R, 6), lambda i: (0, i, 0)),
            full((8, H)), full((D, H)), full((1, H)),
            full((H, H)), full((1, H)),
            full((H, D)), full((1, D)),
        ],
        out_specs=[
            pl.BlockSpec((_R, D), lambda i: (i, 0)),
            pl.BlockSpec((_R, 4), lambda i: (i, 0)),
        ],
        out_shape=[
            jax.ShapeDtypeStruct((N, D), jnp.float32),
            jax.ShapeDtypeStruct((N, 4), jnp.float32),
        ],
    )(x, P, wtop, wpad, b1, w2, b2, w3, b3)


def _tc_final(x, P, wtop, wpad, b1, w2, b2, w3, b3,
              wd1, bd1, wd2, bd2, wd3, bd3, wd4, bd4):
    def body(x_ref, p_ref, wtop_ref, wpad_ref, b1_ref, w2_ref, b2_ref, w3_ref,
             b3_ref, wd1_ref, bd1_ref, wd2_ref, bd2_ref, wd3_ref, bd3_ref,
             wd4_ref, bd4_ref, out_ref):
        xb = x_ref[:]
        aggr = _mean_aggr(xb, p_ref, _R)
        xn = _node_mlp(xb, aggr, wtop_ref, wpad_ref, b1_ref, w2_ref, b2_ref,
                       w3_ref, b3_ref)
        h = jnp.maximum(
            jnp.dot(xn, wd1_ref[:], preferred_element_type=jnp.float32)
            + bd1_ref[:], 0.0)
        h = jnp.maximum(
            jnp.dot(h, wd2_ref[:], preferred_element_type=jnp.float32)
            + bd2_ref[:], 0.0)
        h = jnp.maximum(
            jnp.dot(h, wd3_ref[:], preferred_element_type=jnp.float32)
            + bd3_ref[:], 0.0)
        out = jnp.dot(h, wd4_ref[:], preferred_element_type=jnp.float32)
        out_ref[:] = out + bd4_ref[:]

    full = lambda shape: pl.BlockSpec(shape, lambda i: tuple(0 for _ in shape))
    return pl.pallas_call(
        body,
        grid=(N // _R,),
        in_specs=[
            pl.BlockSpec((_R, D), lambda i: (i, 0)),
            pl.BlockSpec((NC, _R, 6), lambda i: (0, i, 0)),
            full((8, H)), full((D, H)), full((1, H)),
            full((H, H)), full((1, H)),
            full((H, D)), full((1, D)),
            full((D, H)), full((1, H)),
            full((H, H)), full((1, H)),
            full((H, H)), full((1, H)),
            full((H, T)), full((1, T)),
        ],
        out_specs=pl.BlockSpec((_R, T), lambda i: (i, 0)),
        out_shape=jax.ShapeDtypeStruct((N, T), jnp.float32),
    )(x, P, wtop, wpad, b1, w2, b2, w3, b3,
      wd1, bd1, wd2, bd2, wd3, bd3, wd4, bd4)


def kernel(x, edge_index, mode, W2a, b2a, W2b, b2b, W2c, b2c,
           Wd1, bd1, Wd2, bd2, Wd3, bd3, Wd4, bd4):
    row3 = edge_index[0].reshape(NW, NCHUNK, CH)
    col3 = edge_index[1].reshape(NW, NCHUNK, CH)
    feat0 = jnp.concatenate([x[:, 0:3], x[:, 127:128]], axis=1).reshape(-1)
    zeros = jnp.zeros((1, NPAD), jnp.float32)
    wtop = jnp.concatenate([W2a[0:5], jnp.zeros((3, H), jnp.float32)], axis=0)
    wpad = jnp.concatenate([jnp.zeros((3, H), jnp.float32), W2a[5:]], axis=0)
    b1 = b2a.reshape(1, H)
    b2 = b2b.reshape(1, H)
    b3 = b2c.reshape(1, D)

    P1 = jnp.transpose(_edge_sc(row3, col3, feat0, zeros).reshape(6, NC, NPAD)[:, :, :N],
                       (1, 2, 0))
    x1, feat1 = _tc_layer(x, P1, wtop, wpad, b1, W2b, b2, W2c, b3)
    P2 = jnp.transpose(
        _edge_sc(row3, col3, feat1.reshape(-1), zeros).reshape(6, NC, NPAD)[:, :, :N],
        (1, 2, 0))
    out = _tc_final(x1, P2, wtop, wpad, b1, W2b, b2, W2c, b3,
                    Wd1, bd1.reshape(1, H), Wd2, bd2.reshape(1, H),
                    Wd3, bd3.reshape(1, H), Wd4, bd4.reshape(1, T))
    return out


# speedup vs baseline: 23.9445x; 1.2672x over previous
import functools

import jax
import jax.numpy as jnp
from jax import lax
from jax.experimental import pallas as pl
from jax.experimental.pallas import tpu as pltpu
from jax.experimental.pallas import tpu_sc as plsc

N = 10000
E = 320000
D = 128
H = 128
T = 3

NC = 2
NS = 16
NW = NC * NS
EPW = E // NW
CH = 128
EPT = 10240
NCHUNK = EPT // CH
NV = CH // 16
NBUF = 4
NPAD = 10240
RPT = NPAD // NS


def _rsqrt16(q):
    i = plsc.bitcast(q, jnp.int32)
    i = jnp.int32(0x5F3759DF) - lax.shift_right_logical(i, 1)
    y = plsc.bitcast(i, jnp.float32)
    for _ in range(3):
        y = y * (1.5 - 0.5 * q * y * y)
    return y


def _build_edge_sc():
    mesh = plsc.VectorSubcoreMesh(core_axis_name="c", subcore_axis_name="s")
    return functools.partial(
        pl.kernel,
        mesh=mesh,
        compiler_params=pltpu.CompilerParams(needs_layout_passes=False),
        out_type=jax.ShapeDtypeStruct((6, NC, 1, NPAD), jnp.float32),
        scratch_types=[
            pltpu.VMEM((NCHUNK, CH), jnp.int32),
            pltpu.VMEM((NCHUNK, CH), jnp.int32),
            pltpu.VMEM((N * 4,), jnp.float32),
            pltpu.VMEM((NBUF, CH), jnp.float32),
            pltpu.VMEM((NBUF, CH), jnp.float32),
            pltpu.VMEM((NBUF, CH), jnp.float32),
            pltpu.VMEM((NBUF, CH), jnp.float32),
            pltpu.VMEM((NBUF, CH), jnp.float32),
            pltpu.VMEM((NBUF, CH), jnp.float32),
            pltpu.VMEM_SHARED((NPAD,), jnp.float32),
            pltpu.VMEM_SHARED((NPAD,), jnp.float32),
            pltpu.VMEM_SHARED((NPAD,), jnp.float32),
            pltpu.VMEM_SHARED((NPAD,), jnp.float32),
            pltpu.VMEM_SHARED((NPAD,), jnp.float32),
            pltpu.VMEM_SHARED((NPAD,), jnp.float32),
            pltpu.SemaphoreType.DMA,
        ],
    )(_edge_sc_body)


_EDGE_SC = None


def _edge_sc(row3, col3, feat_flat, zeros):
    global _EDGE_SC
    if _EDGE_SC is None:
        _EDGE_SC = _build_edge_sc()
    return _EDGE_SC(row3, col3, feat_flat, zeros)


def _edge_sc_body(row_hbm, col_hbm, feat_hbm, zero_hbm, out_hbm,
                  row_v, col_v, feat_v, m0, m1, m2, m3, m4, m5,
                  a0, a1, a2, a3, a4, a5, sem):
    c = lax.axis_index("c")
    s = lax.axis_index("s")
    wid = c * NS + s
    msgs = (m0, m1, m2, m3, m4)
    accs = (a0, a1, a2, a3, a4, a5)

    pltpu.sync_copy(row_hbm.at[wid], row_v)
    pltpu.sync_copy(col_hbm.at[wid], col_v)
    pltpu.sync_copy(feat_hbm, feat_v)
    for acc in accs:
        pltpu.sync_copy(zero_hbm.at[0, pl.ds(s * RPT, RPT)],
                        acc.at[pl.ds(s * RPT, RPT)])

    ones16 = jnp.full((16,), 1.0, jnp.float32)
    for b in range(NBUF):
        for g in range(NV):
            m5[b, pl.ds(g * 16, 16)] = ones16

    plsc.subcore_barrier()

    def compute_chunk(j, b):
        for v in range(NV):
            r16 = row_v[j, pl.ds(v * 16, 16)]
            c16 = col_v[j, pl.ds(v * 16, 16)]
            bs = lax.shift_left(r16, 2)
            bd = lax.shift_left(c16, 2)
            s0 = plsc.load_gather(feat_v, [bs])
            s1 = plsc.load_gather(feat_v, [bs + 1])
            s2 = plsc.load_gather(feat_v, [bs + 2])
            sf = plsc.load_gather(feat_v, [bs + 3])
            d0 = plsc.load_gather(feat_v, [bd])
            d1 = plsc.load_gather(feat_v, [bd + 1])
            d2 = plsc.load_gather(feat_v, [bd + 2])
            u0 = d0 - s0
            u1 = d1 - s1
            u2 = d2 - s2
            q = u0 * u0 + u1 * u1 + u2 * u2
            nrm = jnp.where(q > 0.0, q * _rsqrt16(q), 0.0)
            sl = pl.ds(v * 16, 16)
            m0[b, sl] = s0
            m1[b, sl] = s1
            m2[b, sl] = s2
            m3[b, sl] = sf
            m4[b, sl] = nrm

    def quad_body(qq, carry):
        handles = []
        for b in range(NBUF):
            j = NBUF * qq + b
            compute_chunk(j, b)
            idx = col_v.at[j]
            for m_, a_ in zip(msgs, accs[:5]):
                handles.append(
                    pltpu.async_copy(m_.at[b], a_.at[idx], sem, add=True))
            handles.append(
                pltpu.async_copy(m5.at[b], a5.at[idx], sem, add=True))
        for h in handles:
            h.wait()
        return carry

    lax.fori_loop(0, NCHUNK // NBUF, quad_body, 0)

    plsc.subcore_barrier()

    for cc, acc in enumerate(accs):
        pltpu.sync_copy(acc.at[pl.ds(s * RPT, RPT)],
                        out_hbm.at[cc, c, 0, pl.ds(s * RPT, RPT)])


def _mean_aggr(xb, p_ref, rows):
    A = p_ref[0] + p_ref[1]
    cnt = A[:, 5:6]
    denom = jnp.maximum(cnt, 1.0)
    disp = cnt * xb[:, 0:3] - A[:, 0:3]
    fagg = cnt * xb[:, 127:128] - A[:, 3:4]
    nrm = A[:, 4:5]
    aggr = jnp.concatenate(
        [disp, nrm, fagg, jnp.zeros((rows, 3), jnp.float32)], axis=1)
    return aggr / denom


def _node_mlp(xb, aggr, wtop_ref, wpad_ref, b1_ref, w2_ref, b2_ref, w3_ref,
              b3_ref):
    h = jnp.dot(aggr, wtop_ref[:], preferred_element_type=jnp.float32)
    h = h + jnp.dot(xb, wpad_ref[:], preferred_element_type=jnp.float32)
    h = jnp.maximum(h + b1_ref[:], 0.0)
    h = jnp.maximum(
        jnp.dot(h, w2_ref[:], preferred_element_type=jnp.float32) + b2_ref[:],
        0.0)
    res = jnp.dot(h, w3_ref[:], preferred_element_type=jnp.float32) + b3_ref[:]
    return xb + jnp.maximum(res, 0.0)


_R = 1000


def _tc_layer(x, P, wtop, wpad, b1, w2, b2, w3, b3):
    def body(x_ref, p_ref, wtop_ref, wpad_ref, b1_ref, w2_ref, b2_ref, w3_ref,
             b3_ref, xo_ref, feat_ref):
        xb = x_ref[:]
        aggr = _mean_aggr(xb, p_ref, _R)
        xn = _node_mlp(xb, aggr, wtop_ref, wpad_ref, b1_ref, w2_ref, b2_ref,
                       w3_ref, b3_ref)
        xo_ref[:] = xn
        feat_ref[:] = jnp.concatenate([xn[:, 0:3], xn[:, 127:128]], axis=1)

    full = lambda shape: pl.BlockSpec(shape, lambda i: tuple(0 for _ in shape))
    return pl.pallas_call(
        body,
        grid=(N // _R,),
        in_specs=[
            pl.BlockSpec((_R, D), lambda i: (i, 0)),
            pl.BlockSpec((NC, _R, 6), lambda i: (0, i, 0)),
            full((8, H)), full((D, H)), full((1, H)),
            full((H, H)), full((1, H)),
            full((H, D)), full((1, D)),
        ],
        out_specs=[
            pl.BlockSpec((_R, D), lambda i: (i, 0)),
            pl.BlockSpec((_R, 4), lambda i: (i, 0)),
        ],
        out_shape=[
            jax.ShapeDtypeStruct((N, D), jnp.float32),
            jax.ShapeDtypeStruct((N, 4), jnp.float32),
        ],
    )(x, P, wtop, wpad, b1, w2, b2, w3, b3)


def _tc_final(x, P, wtop, wpad, b1, w2, b2, w3, b3,
              wd1, bd1, wd2, bd2, wd3, bd3, wd4, bd4):
    def body(x_ref, p_ref, wtop_ref, wpad_ref, b1_ref, w2_ref, b2_ref, w3_ref,
             b3_ref, wd1_ref, bd1_ref, wd2_ref, bd2_ref, wd3_ref, bd3_ref,
             wd4_ref, bd4_ref, out_ref):
        xb = x_ref[:]
        aggr = _mean_aggr(xb, p_ref, _R)
        xn = _node_mlp(xb, aggr, wtop_ref, wpad_ref, b1_ref, w2_ref, b2_ref,
                       w3_ref, b3_ref)
        h = jnp.maximum(
            jnp.dot(xn, wd1_ref[:], preferred_element_type=jnp.float32)
            + bd1_ref[:], 0.0)
        h = jnp.maximum(
            jnp.dot(h, wd2_ref[:], preferred_element_type=jnp.float32)
            + bd2_ref[:], 0.0)
        h = jnp.maximum(
            jnp.dot(h, wd3_ref[:], preferred_element_type=jnp.float32)
            + bd3_ref[:], 0.0)
        out = jnp.dot(h, wd4_ref[:], preferred_element_type=jnp.float32)
        out_ref[:] = out + bd4_ref[:]

    full = lambda shape: pl.BlockSpec(shape, lambda i: tuple(0 for _ in shape))
    return pl.pallas_call(
        body,
        grid=(N // _R,),
        in_specs=[
            pl.BlockSpec((_R, D), lambda i: (i, 0)),
            pl.BlockSpec((NC, _R, 6), lambda i: (0, i, 0)),
            full((8, H)), full((D, H)), full((1, H)),
            full((H, H)), full((1, H)),
            full((H, D)), full((1, D)),
            full((D, H)), full((1, H)),
            full((H, H)), full((1, H)),
            full((H, H)), full((1, H)),
            full((H, T)), full((1, T)),
        ],
        out_specs=pl.BlockSpec((_R, T), lambda i: (i, 0)),
        out_shape=jax.ShapeDtypeStruct((N, T), jnp.float32),
    )(x, P, wtop, wpad, b1, w2, b2, w3, b3,
      wd1, bd1, wd2, bd2, wd3, bd3, wd4, bd4)


def kernel(x, edge_index, mode, W2a, b2a, W2b, b2b, W2c, b2c,
           Wd1, bd1, Wd2, bd2, Wd3, bd3, Wd4, bd4):
    pad_r = jnp.zeros((NW, EPT - EPW), jnp.int32)
    pad_c = jnp.full((NW, EPT - EPW), N, jnp.int32)
    row3 = jnp.concatenate([edge_index[0].reshape(NW, EPW), pad_r],
                           axis=1).reshape(NW, NCHUNK, CH)
    col3 = jnp.concatenate([edge_index[1].reshape(NW, EPW), pad_c],
                           axis=1).reshape(NW, NCHUNK, CH)
    feat0 = jnp.concatenate([x[:, 0:3], x[:, 127:128]], axis=1).reshape(-1)
    zeros = jnp.zeros((1, NPAD), jnp.float32)
    wtop = jnp.concatenate([W2a[0:5], jnp.zeros((3, H), jnp.float32)], axis=0)
    wpad = jnp.concatenate([jnp.zeros((3, H), jnp.float32), W2a[5:]], axis=0)
    b1 = b2a.reshape(1, H)
    b2 = b2b.reshape(1, H)
    b3 = b2c.reshape(1, D)

    P1 = jnp.transpose(
        _edge_sc(row3, col3, feat0, zeros).reshape(6, NC, NPAD)[:, :, :N],
        (1, 2, 0))
    x1, feat1 = _tc_layer(x, P1, wtop, wpad, b1, W2b, b2, W2c, b3)
    P2 = jnp.transpose(
        _edge_sc(row3, col3, feat1.reshape(-1), zeros).reshape(6, NC, NPAD)[:, :, :N],
        (1, 2, 0))
    out = _tc_final(x1, P2, wtop, wpad, b1, W2b, b2, W2c, b3,
                    Wd1, bd1.reshape(1, H), Wd2, bd2.reshape(1, H),
                    Wd3, bd3.reshape(1, H), Wd4, bd4.reshape(1, T))
    return out


# speedup vs baseline: 26.0980x; 1.0899x over previous
import functools

import jax
import jax.numpy as jnp
from jax import lax
from jax.experimental import pallas as pl
from jax.experimental.pallas import tpu as pltpu
from jax.experimental.pallas import tpu_sc as plsc

N = 10000
E = 320000
D = 128
H = 128
T = 3

NC = 2
NS = 16
NW = NC * NS
EPW = E // NW
CH = 128
EPT = 10240
NCHUNK = EPT // CH
NV = CH // 16
NBUF = 8
NPAD = 10240
RPT = NPAD // NS


def _rsqrt16(q):
    i = plsc.bitcast(q, jnp.int32)
    i = jnp.int32(0x5F3759DF) - lax.shift_right_logical(i, 1)
    y = plsc.bitcast(i, jnp.float32)
    for _ in range(2):
        y = y * (1.5 - 0.5 * q * y * y)
    return y


def _make_edge_body(has_cnt):
    def body(row_hbm, col_hbm, feat_hbm, zero_hbm, out_hbm, *rest):
        if has_cnt:
            (row_v, col_v, f0v, f1v, f2v, f3v,
             m0, m1, m2, m3, m4, m5, a0, a1, a2, a3, a4, a5, sem) = rest
            msgs = (m0, m1, m2, m3, m4, m5)
            accs = (a0, a1, a2, a3, a4, a5)
        else:
            (row_v, col_v, f0v, f1v, f2v, f3v,
             m0, m1, m2, m3, m4, a0, a1, a2, a3, a4, sem) = rest
            msgs = (m0, m1, m2, m3, m4)
            accs = (a0, a1, a2, a3, a4)
        c = lax.axis_index("c")
        s = lax.axis_index("s")
        wid = c * NS + s

        pltpu.sync_copy(row_hbm.at[wid], row_v)
        pltpu.sync_copy(col_hbm.at[wid], col_v)
        for k, fkv in enumerate((f0v, f1v, f2v, f3v)):
            pltpu.sync_copy(feat_hbm.at[k, 0], fkv)
        for acc in accs:
            pltpu.sync_copy(zero_hbm.at[0, pl.ds(s * RPT, RPT)],
                            acc.at[pl.ds(s * RPT, RPT)])

        if has_cnt:
            ones16 = jnp.full((16,), 1.0, jnp.float32)
            for b in range(NBUF):
                for g in range(NV):
                    m5[b, pl.ds(g * 16, 16)] = ones16

        plsc.subcore_barrier()

        def compute_chunk(j, b):
            for v in range(NV):
                r16 = row_v[j, pl.ds(v * 16, 16)]
                c16 = col_v[j, pl.ds(v * 16, 16)]
                s0 = plsc.load_gather(f0v, [r16])
                s1 = plsc.load_gather(f1v, [r16])
                s2 = plsc.load_gather(f2v, [r16])
                sf = plsc.load_gather(f3v, [r16])
                d0 = plsc.load_gather(f0v, [c16])
                d1 = plsc.load_gather(f1v, [c16])
                d2 = plsc.load_gather(f2v, [c16])
                u0 = d0 - s0
                u1 = d1 - s1
                u2 = d2 - s2
                q = u0 * u0 + u1 * u1 + u2 * u2
                nrm = q * _rsqrt16(q)
                sl = pl.ds(v * 16, 16)
                m0[b, sl] = s0
                m1[b, sl] = s1
                m2[b, sl] = s2
                m3[b, sl] = sf
                m4[b, sl] = nrm

        def iter_body(qq, carry):
            handles = []
            for b in range(NBUF):
                j = NBUF * qq + b
                compute_chunk(j, b)
                idx = col_v.at[j]
                for m_, a_ in zip(msgs, accs):
                    handles.append(
                        pltpu.async_copy(m_.at[b], a_.at[idx], sem, add=True))
            for h in handles:
                h.wait()
            return carry

        lax.fori_loop(0, NCHUNK // NBUF, iter_body, 0)

        plsc.subcore_barrier()

        for cc, acc in enumerate(accs):
            pltpu.sync_copy(acc.at[pl.ds(s * RPT, RPT)],
                            out_hbm.at[cc, c, 0, pl.ds(s * RPT, RPT)])

    return body


def _build_edge_sc(has_cnt):
    ncomp = 6 if has_cnt else 5
    mesh = plsc.VectorSubcoreMesh(core_axis_name="c", subcore_axis_name="s")
    scratch = (
        [pltpu.VMEM((NCHUNK, CH), jnp.int32)] * 2
        + [pltpu.VMEM((N,), jnp.float32)] * 4
        + [pltpu.VMEM((NBUF, CH), jnp.float32)] * ncomp
        + [pltpu.VMEM_SHARED((NPAD,), jnp.float32)] * ncomp
        + [pltpu.SemaphoreType.DMA]
    )
    return functools.partial(
        pl.kernel,
        mesh=mesh,
        compiler_params=pltpu.CompilerParams(needs_layout_passes=False),
        out_type=jax.ShapeDtypeStruct((ncomp, NC, 1, NPAD), jnp.float32),
        scratch_types=scratch,
    )(_make_edge_body(has_cnt))


_EDGE_SC = {}


def _edge_sc(row3, col3, feat_soa, zeros, has_cnt):
    if has_cnt not in _EDGE_SC:
        _EDGE_SC[has_cnt] = _build_edge_sc(has_cnt)
    return _EDGE_SC[has_cnt](row3, col3, feat_soa, zeros)


def _mean_aggr(xb, A5, cnt, rows):
    denom = jnp.maximum(cnt, 1.0)
    disp = cnt * xb[:, 0:3] - A5[:, 0:3]
    fagg = cnt * xb[:, 127:128] - A5[:, 3:4]
    nrm = A5[:, 4:5]
    aggr = jnp.concatenate(
        [disp, nrm, fagg, jnp.zeros((rows, 3), jnp.float32)], axis=1)
    return aggr / denom


def _node_mlp(xb, aggr, wtop_ref, wpad_ref, b1_ref, w2_ref, b2_ref, w3_ref,
              b3_ref):
    h = jnp.dot(aggr, wtop_ref[:], preferred_element_type=jnp.float32)
    h = h + jnp.dot(xb, wpad_ref[:], preferred_element_type=jnp.float32)
    h = jnp.maximum(h + b1_ref[:], 0.0)
    h = jnp.maximum(
        jnp.dot(h, w2_ref[:], preferred_element_type=jnp.float32) + b2_ref[:],
        0.0)
    res = jnp.dot(h, w3_ref[:], preferred_element_type=jnp.float32) + b3_ref[:]
    return xb + jnp.maximum(res, 0.0)


_R = 1000


def _tc_layer(x, P, wtop, wpad, b1, w2, b2, w3, b3):
    def body(x_ref, p_ref, wtop_ref, wpad_ref, b1_ref, w2_ref, b2_ref, w3_ref,
             b3_ref, xo_ref, feat_ref):
        xb = x_ref[:]
        A = p_ref[0] + p_ref[1]
        aggr = _mean_aggr(xb, A[:, 0:5], A[:, 5:6], _R)
        xn = _node_mlp(xb, aggr, wtop_ref, wpad_ref, b1_ref, w2_ref, b2_ref,
                       w3_ref, b3_ref)
        xo_ref[:] = xn
        feat_ref[:] = jnp.concatenate([xn[:, 0:3], xn[:, 127:128]], axis=1)

    full = lambda shape: pl.BlockSpec(shape, lambda i: tuple(0 for _ in shape))
    return pl.pallas_call(
        body,
        grid=(N // _R,),
        in_specs=[
            pl.BlockSpec((_R, D), lambda i: (i, 0)),
            pl.BlockSpec((NC, _R, 6), lambda i: (0, i, 0)),
            full((8, H)), full((D, H)), full((1, H)),
            full((H, H)), full((1, H)),
            full((H, D)), full((1, D)),
        ],
        out_specs=[
            pl.BlockSpec((_R, D), lambda i: (i, 0)),
            pl.BlockSpec((_R, 4), lambda i: (i, 0)),
        ],
        out_shape=[
            jax.ShapeDtypeStruct((N, D), jnp.float32),
            jax.ShapeDtypeStruct((N, 4), jnp.float32),
        ],
    )(x, P, wtop, wpad, b1, w2, b2, w3, b3)


def _tc_final(x, P1, P2, wtop, wpad, b1, w2, b2, w3, b3,
              wd1, bd1, wd2, bd2, wd3, bd3, wd4, bd4):
    def body(x_ref, p1_ref, p2_ref, wtop_ref, wpad_ref, b1_ref, w2_ref,
             b2_ref, w3_ref, b3_ref, wd1_ref, bd1_ref, wd2_ref, bd2_ref,
             wd3_ref, bd3_ref, wd4_ref, bd4_ref, out_ref):
        xb = x_ref[:]
        A5 = p2_ref[0] + p2_ref[1]
        cnt = p1_ref[0, :, 5:6] + p1_ref[1, :, 5:6]
        aggr = _mean_aggr(xb, A5, cnt, _R)
        xn = _node_mlp(xb, aggr, wtop_ref, wpad_ref, b1_ref, w2_ref, b2_ref,
                       w3_ref, b3_ref)
        h = jnp.maximum(
            jnp.dot(xn, wd1_ref[:], preferred_element_type=jnp.float32)
            + bd1_ref[:], 0.0)
        h = jnp.maximum(
            jnp.dot(h, wd2_ref[:], preferred_element_type=jnp.float32)
            + bd2_ref[:], 0.0)
        h = jnp.maximum(
            jnp.dot(h, wd3_ref[:], preferred_element_type=jnp.float32)
            + bd3_ref[:], 0.0)
        out = jnp.dot(h, wd4_ref[:], preferred_element_type=jnp.float32)
        out_ref[:] = out + bd4_ref[:]

    full = lambda shape: pl.BlockSpec(shape, lambda i: tuple(0 for _ in shape))
    return pl.pallas_call(
        body,
        grid=(N // _R,),
        in_specs=[
            pl.BlockSpec((_R, D), lambda i: (i, 0)),
            pl.BlockSpec((NC, _R, 6), lambda i: (0, i, 0)),
            pl.BlockSpec((NC, _R, 5), lambda i: (0, i, 0)),
            full((8, H)), full((D, H)), full((1, H)),
            full((H, H)), full((1, H)),
            full((H, D)), full((1, D)),
            full((D, H)), full((1, H)),
            full((H, H)), full((1, H)),
            full((H, H)), full((1, H)),
            full((H, T)), full((1, T)),
        ],
        out_specs=pl.BlockSpec((_R, T), lambda i: (i, 0)),
        out_shape=jax.ShapeDtypeStruct((N, T), jnp.float32),
    )(x, P1, P2, wtop, wpad, b1, w2, b2, w3, b3,
      wd1, bd1, wd2, bd2, wd3, bd3, wd4, bd4)


def kernel(x, edge_index, mode, W2a, b2a, W2b, b2b, W2c, b2c,
           Wd1, bd1, Wd2, bd2, Wd3, bd3, Wd4, bd4):
    pad_r = jnp.zeros((NW, EPT - EPW), jnp.int32)
    pad_c = jnp.full((NW, EPT - EPW), N, jnp.int32)
    row3 = jnp.concatenate([edge_index[0].reshape(NW, EPW), pad_r],
                           axis=1).reshape(NW, NCHUNK, CH)
    col3 = jnp.concatenate([edge_index[1].reshape(NW, EPW), pad_c],
                           axis=1).reshape(NW, NCHUNK, CH)
    feat0 = jnp.stack(
        [x[:, 0], x[:, 1], x[:, 2], x[:, 127]], axis=0).reshape(4, 1, N)
    zeros = jnp.zeros((1, NPAD), jnp.float32)
    wtop = jnp.concatenate([W2a[0:5], jnp.zeros((3, H), jnp.float32)], axis=0)
    wpad = jnp.concatenate([jnp.zeros((3, H), jnp.float32), W2a[5:]], axis=0)
    b1 = b2a.reshape(1, H)
    b2 = b2b.reshape(1, H)
    b3 = b2c.reshape(1, D)

    P1 = jnp.transpose(
        _edge_sc(row3, col3, feat0, zeros, True)
        .reshape(6, NC, NPAD)[:, :, :N], (1, 2, 0))
    x1, feat1 = _tc_layer(x, P1, wtop, wpad, b1, W2b, b2, W2c, b3)
    feat1_soa = jnp.transpose(feat1).reshape(4, 1, N)
    P2 = jnp.transpose(
        _edge_sc(row3, col3, feat1_soa, zeros, False)
        .reshape(5, NC, NPAD)[:, :, :N], (1, 2, 0))
    out = _tc_final(x1, P1, P2, wtop, wpad, b1, W2b, b2, W2c, b3,
                    Wd1, bd1.reshape(1, H), Wd2, bd2.reshape(1, H),
                    Wd3, bd3.reshape(1, H), Wd4, bd4.reshape(1, T))
    return out


# speedup vs baseline: 28.1926x; 1.0803x over previous
import functools

import jax
import jax.numpy as jnp
from jax import lax
from jax.experimental import pallas as pl
from jax.experimental.pallas import tpu as pltpu
from jax.experimental.pallas import tpu_sc as plsc

N = 10000
E = 320000
D = 128
H = 128
T = 3

NC = 2
NS = 16
NW = NC * NS
EPW = E // NW
CH = 80
NCHUNK = EPW // CH
NV = CH // 16
NBUF = 25
NPAD = 10240
RPT = NPAD // NS


def _rsqrt16(q):
    i = plsc.bitcast(q, jnp.int32)
    i = jnp.int32(0x5F3759DF) - lax.shift_right_logical(i, 1)
    y = plsc.bitcast(i, jnp.float32)
    for _ in range(2):
        y = y * (1.5 - 0.5 * q * y * y)
    return y


def _make_edge_body(has_cnt):
    def body(row_hbm, col_hbm, feat_hbm, zero_hbm, out_hbm, *rest):
        if has_cnt:
            (row_v, col_v, f0v, f1v, f2v, f3v,
             m0, m1, m2, m3, m4, m5, a0, a1, a2, a3, a4, a5, sem) = rest
            msgs = (m0, m1, m2, m3, m4, m5)
            accs = (a0, a1, a2, a3, a4, a5)
        else:
            (row_v, col_v, f0v, f1v, f2v, f3v,
             m0, m1, m2, m3, m4, a0, a1, a2, a3, a4, sem) = rest
            msgs = (m0, m1, m2, m3, m4)
            accs = (a0, a1, a2, a3, a4)
        c = lax.axis_index("c")
        s = lax.axis_index("s")
        wid = c * NS + s

        pltpu.sync_copy(row_hbm.at[wid], row_v)
        pltpu.sync_copy(col_hbm.at[wid], col_v)
        for k, fkv in enumerate((f0v, f1v, f2v, f3v)):
            pltpu.sync_copy(feat_hbm.at[k, 0], fkv)
        for acc in accs:
            pltpu.sync_copy(zero_hbm.at[0, pl.ds(s * RPT, RPT)],
                            acc.at[pl.ds(s * RPT, RPT)])

        if has_cnt:
            ones16 = jnp.full((16,), 1.0, jnp.float32)
            for b in range(NBUF):
                for g in range(NV):
                    m5[b, pl.ds(g * 16, 16)] = ones16

        plsc.subcore_barrier()

        def compute_chunk(j, b):
            for v in range(NV):
                r16 = row_v[j, pl.ds(v * 16, 16)]
                c16 = col_v[j, pl.ds(v * 16, 16)]
                s0 = plsc.load_gather(f0v, [r16])
                s1 = plsc.load_gather(f1v, [r16])
                s2 = plsc.load_gather(f2v, [r16])
                sf = plsc.load_gather(f3v, [r16])
                d0 = plsc.load_gather(f0v, [c16])
                d1 = plsc.load_gather(f1v, [c16])
                d2 = plsc.load_gather(f2v, [c16])
                u0 = d0 - s0
                u1 = d1 - s1
                u2 = d2 - s2
                q = u0 * u0 + u1 * u1 + u2 * u2
                nrm = q * _rsqrt16(q)
                sl = pl.ds(v * 16, 16)
                m0[b, sl] = s0
                m1[b, sl] = s1
                m2[b, sl] = s2
                m3[b, sl] = sf
                m4[b, sl] = nrm

        def iter_body(qq, carry):
            handles = []
            for b in range(NBUF):
                j = NBUF * qq + b
                compute_chunk(j, b)
                idx = col_v.at[j]
                for m_, a_ in zip(msgs, accs):
                    handles.append(
                        pltpu.async_copy(m_.at[b], a_.at[idx], sem, add=True))
            for h in handles:
                h.wait()
            return carry

        lax.fori_loop(0, NCHUNK // NBUF, iter_body, 0)

        plsc.subcore_barrier()

        for cc, acc in enumerate(accs):
            pltpu.sync_copy(acc.at[pl.ds(s * RPT, RPT)],
                            out_hbm.at[cc, c, 0, pl.ds(s * RPT, RPT)])

    return body


def _build_edge_sc(has_cnt):
    ncomp = 6 if has_cnt else 5
    mesh = plsc.VectorSubcoreMesh(core_axis_name="c", subcore_axis_name="s")
    scratch = (
        [pltpu.VMEM((NCHUNK, CH), jnp.int32)] * 2
        + [pltpu.VMEM((N,), jnp.float32)] * 4
        + [pltpu.VMEM((NBUF, CH), jnp.float32)] * ncomp
        + [pltpu.VMEM_SHARED((NPAD,), jnp.float32)] * ncomp
        + [pltpu.SemaphoreType.DMA]
    )
    return functools.partial(
        pl.kernel,
        mesh=mesh,
        compiler_params=pltpu.CompilerParams(needs_layout_passes=False),
        out_type=jax.ShapeDtypeStruct((ncomp, NC, 1, NPAD), jnp.float32),
        scratch_types=scratch,
    )(_make_edge_body(has_cnt))


_EDGE_SC = {}


def _edge_sc(row3, col3, feat_soa, zeros, has_cnt):
    if has_cnt not in _EDGE_SC:
        _EDGE_SC[has_cnt] = _build_edge_sc(has_cnt)
    return _EDGE_SC[has_cnt](row3, col3, feat_soa, zeros)


def _mean_aggr(xb, A5, cnt, rows):
    denom = jnp.maximum(cnt, 1.0)
    disp = cnt * xb[:, 0:3] - A5[:, 0:3]
    fagg = cnt * xb[:, 127:128] - A5[:, 3:4]
    nrm = A5[:, 4:5]
    aggr = jnp.concatenate(
        [disp, nrm, fagg, jnp.zeros((rows, 3), jnp.float32)], axis=1)
    return aggr / denom


def _node_mlp(xb, aggr, wtop_ref, wpad_ref, b1_ref, w2_ref, b2_ref, w3_ref,
              b3_ref):
    h = jnp.dot(aggr, wtop_ref[:], preferred_element_type=jnp.float32)
    h = h + jnp.dot(xb, wpad_ref[:], preferred_element_type=jnp.float32)
    h = jnp.maximum(h + b1_ref[:], 0.0)
    h = jnp.maximum(
        jnp.dot(h, w2_ref[:], preferred_element_type=jnp.float32) + b2_ref[:],
        0.0)
    res = jnp.dot(h, w3_ref[:], preferred_element_type=jnp.float32) + b3_ref[:]
    return xb + jnp.maximum(res, 0.0)


_R = 2000


def _tc_layer(x, P, wtop, wpad, b1, w2, b2, w3, b3):
    def body(x_ref, p_ref, wtop_ref, wpad_ref, b1_ref, w2_ref, b2_ref, w3_ref,
             b3_ref, xo_ref, feat_ref):
        xb = x_ref[:]
        A = p_ref[0] + p_ref[1]
        aggr = _mean_aggr(xb, A[:, 0:5], A[:, 5:6], _R)
        xn = _node_mlp(xb, aggr, wtop_ref, wpad_ref, b1_ref, w2_ref, b2_ref,
                       w3_ref, b3_ref)
        xo_ref[:] = xn
        feat_ref[:] = jnp.concatenate([xn[:, 0:3], xn[:, 127:128]], axis=1)

    full = lambda shape: pl.BlockSpec(shape, lambda i: tuple(0 for _ in shape))
    return pl.pallas_call(
        body,
        grid=(N // _R,),
        in_specs=[
            pl.BlockSpec((_R, D), lambda i: (i, 0)),
            pl.BlockSpec((NC, _R, 6), lambda i: (0, i, 0)),
            full((8, H)), full((D, H)), full((1, H)),
            full((H, H)), full((1, H)),
            full((H, D)), full((1, D)),
        ],
        out_specs=[
            pl.BlockSpec((_R, D), lambda i: (i, 0)),
            pl.BlockSpec((_R, 4), lambda i: (i, 0)),
        ],
        out_shape=[
            jax.ShapeDtypeStruct((N, D), jnp.float32),
            jax.ShapeDtypeStruct((N, 4), jnp.float32),
        ],
    )(x, P, wtop, wpad, b1, w2, b2, w3, b3)


def _tc_final(x, P1, P2, wtop, wpad, b1, w2, b2, w3, b3,
              wd1, bd1, wd2, bd2, wd3, bd3, wd4, bd4):
    def body(x_ref, p1_ref, p2_ref, wtop_ref, wpad_ref, b1_ref, w2_ref,
             b2_ref, w3_ref, b3_ref, wd1_ref, bd1_ref, wd2_ref, bd2_ref,
             wd3_ref, bd3_ref, wd4_ref, bd4_ref, out_ref):
        xb = x_ref[:]
        A5 = p2_ref[0] + p2_ref[1]
        cnt = p1_ref[0, :, 5:6] + p1_ref[1, :, 5:6]
        aggr = _mean_aggr(xb, A5, cnt, _R)
        xn = _node_mlp(xb, aggr, wtop_ref, wpad_ref, b1_ref, w2_ref, b2_ref,
                       w3_ref, b3_ref)
        h = jnp.maximum(
            jnp.dot(xn, wd1_ref[:], preferred_element_type=jnp.float32)
            + bd1_ref[:], 0.0)
        h = jnp.maximum(
            jnp.dot(h, wd2_ref[:], preferred_element_type=jnp.float32)
            + bd2_ref[:], 0.0)
        h = jnp.maximum(
            jnp.dot(h, wd3_ref[:], preferred_element_type=jnp.float32)
            + bd3_ref[:], 0.0)
        out = jnp.dot(h, wd4_ref[:], preferred_element_type=jnp.float32)
        out_ref[:] = out + bd4_ref[:]

    full = lambda shape: pl.BlockSpec(shape, lambda i: tuple(0 for _ in shape))
    return pl.pallas_call(
        body,
        grid=(N // _R,),
        in_specs=[
            pl.BlockSpec((_R, D), lambda i: (i, 0)),
            pl.BlockSpec((NC, _R, 6), lambda i: (0, i, 0)),
            pl.BlockSpec((NC, _R, 5), lambda i: (0, i, 0)),
            full((8, H)), full((D, H)), full((1, H)),
            full((H, H)), full((1, H)),
            full((H, D)), full((1, D)),
            full((D, H)), full((1, H)),
            full((H, H)), full((1, H)),
            full((H, H)), full((1, H)),
            full((H, T)), full((1, T)),
        ],
        out_specs=pl.BlockSpec((_R, T), lambda i: (i, 0)),
        out_shape=jax.ShapeDtypeStruct((N, T), jnp.float32),
    )(x, P1, P2, wtop, wpad, b1, w2, b2, w3, b3,
      wd1, bd1, wd2, bd2, wd3, bd3, wd4, bd4)


def kernel(x, edge_index, mode, W2a, b2a, W2b, b2b, W2c, b2c,
           Wd1, bd1, Wd2, bd2, Wd3, bd3, Wd4, bd4):
    row3 = edge_index[0].reshape(NW, NCHUNK, CH)
    col3 = edge_index[1].reshape(NW, NCHUNK, CH)
    feat0 = jnp.stack(
        [x[:, 0], x[:, 1], x[:, 2], x[:, 127]], axis=0).reshape(4, 1, N)
    zeros = jnp.zeros((1, NPAD), jnp.float32)
    wtop = jnp.concatenate([W2a[0:5], jnp.zeros((3, H), jnp.float32)], axis=0)
    wpad = jnp.concatenate([jnp.zeros((3, H), jnp.float32), W2a[5:]], axis=0)
    b1 = b2a.reshape(1, H)
    b2 = b2b.reshape(1, H)
    b3 = b2c.reshape(1, D)

    P1 = jnp.transpose(
        _edge_sc(row3, col3, feat0, zeros, True)
        .reshape(6, NC, NPAD)[:, :, :N], (1, 2, 0))
    x1, feat1 = _tc_layer(x, P1, wtop, wpad, b1, W2b, b2, W2c, b3)
    feat1_soa = jnp.transpose(feat1).reshape(4, 1, N)
    P2 = jnp.transpose(
        _edge_sc(row3, col3, feat1_soa, zeros, False)
        .reshape(5, NC, NPAD)[:, :, :N], (1, 2, 0))
    out = _tc_final(x1, P1, P2, wtop, wpad, b1, W2b, b2, W2c, b3,
                    Wd1, bd1.reshape(1, H), Wd2, bd2.reshape(1, H),
                    Wd3, bd3.reshape(1, H), Wd4, bd4.reshape(1, T))
    return out
